# baseline (speedup 1.0000x reference)
import math

import jax
import jax.numpy as jnp
from jax import lax
from jax.experimental import pallas as pl
from jax.experimental.pallas import tpu as pltpu
from jax.experimental.pallas import tpu_sc as plsc

D_MODEL = 512
SCALE = math.sqrt(512.0)

NC = 2
NS = 16
LANES = 16
NW = NC * NS

B_TOTAL = 4 * 8192
B_PER_W = B_TOTAL // NW
CHUNK_MAX = 64
NBUF = 3
DEPTH = 2
VECS_PER_ROW = D_MODEL // LANES

CHUNKS = [8, 16, 32, 56] + [CHUNK_MAX] * 14 + [16]
assert sum(CHUNKS) == B_PER_W
OFFSETS = [sum(CHUNKS[:i]) for i in range(len(CHUNKS))]
N_CHUNKS = len(CHUNKS)


def _body(table_hbm, idx_hbm, out_hbm, idx_v, rows_v, in_sems, out_sems):
    wid = lax.axis_index("s") * NC + lax.axis_index("c")
    base = wid * B_PER_W
    n_batch_w = 8192 // B_PER_W

    pltpu.sync_copy(
        idx_hbm.at[wid // n_batch_w, pl.ds((wid % n_batch_w) * B_PER_W, B_PER_W)],
        idx_v,
    )

    def gather_desc(c, buf):
        return pltpu.make_async_copy(
            table_hbm.at[idx_v.at[pl.ds(OFFSETS[c], CHUNKS[c])]],
            rows_v.at[buf, pl.ds(0, CHUNKS[c])],
            in_sems.at[buf],
        )

    def scatter_desc(c, buf):
        return pltpu.make_async_copy(
            rows_v.at[buf, pl.ds(0, CHUNKS[c])],
            out_hbm.at[pl.ds(base + OFFSETS[c], CHUNKS[c])],
            out_sems.at[buf],
        )

    def scale_buf(c, buf):
        def row_body(r, _):
            for j in range(VECS_PER_ROW):
                sl = slice(j * LANES, (j + 1) * LANES)
                rows_v[buf, r, sl] = rows_v[buf, r, sl] * SCALE
            return 0

        lax.fori_loop(0, CHUNKS[c], row_body, 0)

    for c in range(DEPTH):
        gather_desc(c, c % NBUF).start()
    for c in range(N_CHUNKS):
        buf = c % NBUF
        if c + DEPTH < N_CHUNKS:
            nxt = (c + DEPTH) % NBUF
            prev = c + DEPTH - NBUF
            if prev >= 0:
                scatter_desc(prev, nxt).wait()
            gather_desc(c + DEPTH, nxt).start()
        gather_desc(c, buf).wait()
        scale_buf(c, buf)
        scatter_desc(c, buf).start()
    for c in range(max(N_CHUNKS - NBUF, 0), N_CHUNKS):
        scatter_desc(c, c % NBUF).wait()


@jax.jit
def _embed(table, indices):
    mesh = plsc.VectorSubcoreMesh(core_axis_name="c", subcore_axis_name="s")
    fn = pl.kernel(
        _body,
        out_type=jax.ShapeDtypeStruct((B_TOTAL, D_MODEL), jnp.float32),
        mesh=mesh,
        scratch_types=[
            pltpu.VMEM((B_PER_W,), jnp.int32),
            pltpu.VMEM((NBUF, CHUNK_MAX, D_MODEL), jnp.float32),
            pltpu.SemaphoreType.DMA((NBUF,)),
            pltpu.SemaphoreType.DMA((NBUF,)),
        ],
    )
    return fn(table, indices)


def kernel(indices, table):
    out = _embed(table, indices)
    return out.reshape(indices.shape + (D_MODEL,))


# speedup vs baseline: 1.0104x; 1.0104x over previous
import math

import jax
import jax.numpy as jnp
from jax import lax
from jax.experimental import pallas as pl
from jax.experimental.pallas import tpu as pltpu
from jax.experimental.pallas import tpu_sc as plsc

D_MODEL = 512
SCALE = math.sqrt(512.0)

NC = 2
NS = 16
LANES = 16
NW = NC * NS

B_TOTAL = 4 * 8192
B_PER_W = B_TOTAL // NW
CHUNK_MAX = 64
NBUF = 3
DEPTH = 2
VECS_PER_ROW = D_MODEL // LANES

CHUNKS = [CHUNK_MAX] * 16
assert sum(CHUNKS) == B_PER_W
OFFSETS = [sum(CHUNKS[:i]) for i in range(len(CHUNKS))]
N_CHUNKS = len(CHUNKS)


def _body(table_hbm, idx_hbm, out_hbm, idx_v, rows_v, in_sems, out_sems):
    wid = lax.axis_index("s") * NC + lax.axis_index("c")
    base = wid * B_PER_W
    n_batch_w = 8192 // B_PER_W

    pltpu.sync_copy(
        idx_hbm.at[wid // n_batch_w, pl.ds((wid % n_batch_w) * B_PER_W, B_PER_W)],
        idx_v,
    )

    def gather_desc(c, buf):
        return pltpu.make_async_copy(
            table_hbm.at[idx_v.at[pl.ds(OFFSETS[c], CHUNKS[c])]],
            rows_v.at[buf, pl.ds(0, CHUNKS[c])],
            in_sems.at[buf],
        )

    def scatter_desc(c, buf):
        return pltpu.make_async_copy(
            rows_v.at[buf, pl.ds(0, CHUNKS[c])],
            out_hbm.at[pl.ds(base + OFFSETS[c], CHUNKS[c])],
            out_sems.at[buf],
        )

    def scale_buf(c, buf):
        def row_body(r, _):
            def vec_body(j, _):
                sl = pl.ds(j * LANES, LANES)
                rows_v[buf, r, sl] = rows_v[buf, r, sl] * SCALE
                return 0

            return lax.fori_loop(0, VECS_PER_ROW, vec_body, 0, unroll=8)

        lax.fori_loop(0, CHUNKS[c], row_body, 0)

    for c in range(DEPTH):
        gather_desc(c, c % NBUF).start()
    for c in range(N_CHUNKS):
        buf = c % NBUF
        if c + DEPTH < N_CHUNKS:
            nxt = (c + DEPTH) % NBUF
            prev = c + DEPTH - NBUF
            if prev >= 0:
                scatter_desc(prev, nxt).wait()
            gather_desc(c + DEPTH, nxt).start()
        gather_desc(c, buf).wait()
        scale_buf(c, buf)
        scatter_desc(c, buf).start()
    for c in range(max(N_CHUNKS - NBUF, 0), N_CHUNKS):
        scatter_desc(c, c % NBUF).wait()


@jax.jit
def _embed(table, indices):
    mesh = plsc.VectorSubcoreMesh(core_axis_name="c", subcore_axis_name="s")
    fn = pl.kernel(
        _body,
        out_type=jax.ShapeDtypeStruct((B_TOTAL, D_MODEL), jnp.float32),
        mesh=mesh,
        scratch_types=[
            pltpu.VMEM((B_PER_W,), jnp.int32),
            pltpu.VMEM((NBUF, CHUNK_MAX, D_MODEL), jnp.float32),
            pltpu.SemaphoreType.DMA((NBUF,)),
            pltpu.SemaphoreType.DMA((NBUF,)),
        ],
    )
    return fn(table, indices)


def kernel(indices, table):
    out = _embed(table, indices)
    return out.reshape(indices.shape + (D_MODEL,))


# speedup vs baseline: 1.0774x; 1.0663x over previous
import math

import jax
import jax.numpy as jnp
from jax import lax
from jax.experimental import pallas as pl
from jax.experimental.pallas import tpu as pltpu
from jax.experimental.pallas import tpu_sc as plsc

D_MODEL = 512
SCALE = math.sqrt(512.0)

NC = 2
NS = 16
LANES = 16
NW = NC * NS

B_TOTAL = 4 * 8192
B_PER_W = B_TOTAL // NW
CHUNK_MAX = 64
NBUF = 3
DEPTH = 2
VECS_PER_ROW = D_MODEL // LANES

CHUNKS = [CHUNK_MAX] * 16
assert sum(CHUNKS) == B_PER_W
OFFSETS = [sum(CHUNKS[:i]) for i in range(len(CHUNKS))]
N_CHUNKS = len(CHUNKS)


def _body(table_hbm, idx_hbm, out_hbm, idx_v, rows_v, in_sems, out_sems):
    wid = lax.axis_index("s") * NC + lax.axis_index("c")
    base = wid * B_PER_W
    n_batch_w = 8192 // B_PER_W

    pltpu.sync_copy(
        idx_hbm.at[wid // n_batch_w, pl.ds((wid % n_batch_w) * B_PER_W, B_PER_W)],
        idx_v,
    )

    def gather_desc(c, buf):
        return pltpu.make_async_copy(
            table_hbm.at[idx_v.at[pl.ds(c * CHUNK_MAX, CHUNK_MAX)]],
            rows_v.at[buf],
            in_sems.at[buf],
        )

    def scatter_desc(c, buf):
        return pltpu.make_async_copy(
            rows_v.at[buf],
            out_hbm.at[pl.ds(base + c * CHUNK_MAX, CHUNK_MAX)],
            out_sems.at[buf],
        )

    def scale_buf(c, buf):
        def row_body(r, _):
            def vec_body(j, _):
                sl = pl.ds(j * LANES, LANES)
                rows_v[buf, r, sl] = rows_v[buf, r, sl] * SCALE
                return 0

            return lax.fori_loop(0, VECS_PER_ROW, vec_body, 0, unroll=8)

        lax.fori_loop(0, CHUNK_MAX, row_body, 0)

    for c in range(DEPTH):
        gather_desc(c, c % NBUF).start()

    def chunk_body(c, _):
        buf = lax.rem(c, NBUF)
        nxt = lax.rem(c + DEPTH, NBUF)

        @pl.when(jnp.logical_and(c >= 1, c + DEPTH < N_CHUNKS))
        def _():
            scatter_desc(c - 1, nxt).wait()

        @pl.when(c + DEPTH < N_CHUNKS)
        def _():
            gather_desc(c + DEPTH, nxt).start()

        gather_desc(c, buf).wait()
        scale_buf(c, buf)
        scatter_desc(c, buf).start()
        return 0

    lax.fori_loop(0, N_CHUNKS, chunk_body, 0)
    for c in range(max(N_CHUNKS - NBUF, 0), N_CHUNKS):
        scatter_desc(c, c % NBUF).wait()


@jax.jit
def _embed(table, indices):
    mesh = plsc.VectorSubcoreMesh(core_axis_name="c", subcore_axis_name="s")
    fn = pl.kernel(
        _body,
        out_type=jax.ShapeDtypeStruct((B_TOTAL, D_MODEL), jnp.float32),
        mesh=mesh,
        scratch_types=[
            pltpu.VMEM((B_PER_W,), jnp.int32),
            pltpu.VMEM((NBUF, CHUNK_MAX, D_MODEL), jnp.float32),
            pltpu.SemaphoreType.DMA((NBUF,)),
            pltpu.SemaphoreType.DMA((NBUF,)),
        ],
    )
    return fn(table, indices)


def kernel(indices, table):
    out = _embed(table, indices)
    return out.reshape(indices.shape + (D_MODEL,))
